# baseline (speedup 1.0000x reference)
import functools

import jax
import jax.numpy as jnp
from jax import lax
from jax.experimental import pallas as pl
from jax.experimental.pallas import tpu as pltpu
from jax.experimental.pallas import tpu_sc as plsc

B = 16384
D = 16
NW = 32
BPW = B // NW
NCHUNK = 4
CH = BPW // NCHUNK


def _sc_gather4(uidx, iidx, eu_mlp, ei_mlp, eu_mf, ei_mf):
    info = plsc.get_sparse_core_info()
    nc = info.num_cores
    mesh = plsc.VectorSubcoreMesh(core_axis_name="c", subcore_axis_name="s")
    row_t = jax.ShapeDtypeStruct((NW, NCHUNK, CH, D), jnp.float32)

    @functools.partial(
        pl.kernel,
        mesh=mesh,
        compiler_params=pltpu.CompilerParams(use_tc_tiling_on_sc=False),
        out_type=[row_t, row_t, row_t, row_t],
        scratch_types=[
            pltpu.VMEM((NCHUNK, CH), jnp.int32),
            pltpu.VMEM((NCHUNK, CH), jnp.int32),
            pltpu.VMEM((NCHUNK, CH, D), jnp.float32),
            pltpu.VMEM((NCHUNK, CH, D), jnp.float32),
            pltpu.VMEM((NCHUNK, CH, D), jnp.float32),
            pltpu.VMEM((NCHUNK, CH, D), jnp.float32),
            pltpu.SemaphoreType.DMA,
        ],
    )
    def k(uidx_h, iidx_h, t0, t1, t2, t3, o0, o1, o2, o3,
          uix, iix, r0, r1, r2, r3, sem):
        wid = lax.axis_index("s") * nc + lax.axis_index("c")
        pltpu.sync_copy(uidx_h.at[wid], uix)
        pltpu.sync_copy(iidx_h.at[wid], iix)
        copies = []
        for j in range(NCHUNK):
            copies.append(pltpu.async_copy(t0.at[uix.at[j]], r0.at[j], sem))
            copies.append(pltpu.async_copy(t1.at[iix.at[j]], r1.at[j], sem))
            copies.append(pltpu.async_copy(t2.at[uix.at[j]], r2.at[j], sem))
            copies.append(pltpu.async_copy(t3.at[iix.at[j]], r3.at[j], sem))
        for c in copies:
            c.wait()
        pltpu.sync_copy(r0, o0.at[wid])
        pltpu.sync_copy(r1, o1.at[wid])
        pltpu.sync_copy(r2, o2.at[wid])
        pltpu.sync_copy(r3, o3.at[wid])

    u2 = uidx.reshape(NW, NCHUNK, CH).astype(jnp.int32)
    i2 = iidx.reshape(NW, NCHUNK, CH).astype(jnp.int32)
    return k(u2, i2, eu_mlp, ei_mlp, eu_mf, ei_mf)


def _dense_body(ue, ie, uf, itf, w1u, w1i, b1, w2, b2, wom, wof, bo, out):
    h1 = jnp.maximum(
        jnp.dot(ue[...], w1u[...], preferred_element_type=jnp.float32)
        + jnp.dot(ie[...], w1i[...], preferred_element_type=jnp.float32)
        + b1[...], 0.0)
    h2 = jnp.maximum(
        jnp.dot(h1, w2[...], preferred_element_type=jnp.float32) + b2[...], 0.0)
    mf = uf[...] * itf[...]
    logit = (jnp.dot(h2, wom[...], preferred_element_type=jnp.float32)
             + jnp.dot(mf, wof[...], preferred_element_type=jnp.float32)
             + bo[...])
    out[...] = 1.0 / (1.0 + jnp.exp(-logit))


def _tc_dense(ue, ie, uf, itf, W1, b1, W2, b2, Wo, bo):
    bn = 2048
    grid = B // bn
    row = lambda: pl.BlockSpec((bn, D), lambda i: (i, 0))
    full = lambda a: pl.BlockSpec(a.shape, lambda i: (0,) * a.ndim)
    w1u, w1i = W1[:D], W1[D:]
    wom, wof = Wo[:8], Wo[8:]
    b1r, b2r, bor = b1.reshape(1, -1), b2.reshape(1, -1), bo.reshape(1, 1)
    return pl.pallas_call(
        _dense_body,
        grid=(grid,),
        in_specs=[row(), row(), row(), row(),
                  full(w1u), full(w1i), full(b1r), full(W2), full(b2r),
                  full(wom), full(wof), full(bor)],
        out_specs=pl.BlockSpec((bn, 1), lambda i: (i, 0)),
        out_shape=jax.ShapeDtypeStruct((B, 1), jnp.float32),
    )(ue, ie, uf, itf, w1u, w1i, b1r, W2, b2r, wom, wof, bor)


def kernel(user_indices, item_indices, Eu_mlp, Ei_mlp, Eu_mf, Ei_mf,
           W1, b1, W2, b2, Wo, bo):
    r0, r1, r2, r3 = _sc_gather4(user_indices, item_indices,
                                 Eu_mlp, Ei_mlp, Eu_mf, Ei_mf)
    ue = r0.reshape(B, D)
    ie = r1.reshape(B, D)
    uf = r2.reshape(B, D)
    itf = r3.reshape(B, D)
    return _tc_dense(ue, ie, uf, itf, W1, b1, W2, b2, Wo, bo)
